# baseline (speedup 1.0000x reference)
import functools

import jax
import jax.numpy as jnp
from jax import lax
from jax.experimental import pallas as pl
from jax.experimental.pallas import tpu as pltpu
from jax.experimental.pallas import tpu_sc as plsc

_INFO = plsc.get_sparse_core_info()
_NC, _NS, _L = _INFO.num_cores, _INFO.num_subcores, _INFO.num_lanes
_NW = _NC * _NS


def _sc_body(n, c, d, rw, ch, nchunk, x_hbm, emb_hbm, out_hbm,
             x_a, x_b, idx_v, rows_v, sem_a, sem_b, sem_g):
    wid = lax.axis_index("s") * _NC + lax.axis_index("c")
    flat0 = wid * rw * c
    lanes = lax.iota(jnp.int32, _L)
    neg_inf = jnp.full((_L,), -jnp.inf, jnp.float32)
    zeros_i = jnp.zeros((_L,), jnp.int32)

    bufs = (x_a, x_b)
    sems = (sem_a, sem_b)

    def start_load(chunk, buf):
        return pltpu.make_async_copy(
            x_hbm.at[pl.ds(flat0 + chunk * ch * c, ch * c)],
            bufs[buf], sems[buf])

    start_load(0, 0).start()

    for chunk in range(nchunk):
        buf = chunk % 2
        if chunk + 1 < nchunk:
            start_load(chunk + 1, 1 - buf).start()
        start_load(chunk, buf).wait()

        for g in range(ch // _L):
            lanebase = (lanes + g * _L) * c

            def jbody(j, carry, lanebase=lanebase, buf=buf):
                cm, ci = carry
                jv = jnp.full((_L,), j, jnp.int32)
                v = plsc.load_gather(bufs[buf], [lanebase + jv])
                upd = v > cm
                cm = jnp.where(upd, v, cm)
                ci = jnp.where(upd, jv, ci)
                return cm, ci

            _, ci = lax.fori_loop(0, c, jbody, (neg_inf, zeros_i))
            idx_v[pl.ds(chunk * ch + g * _L, _L)] = ci

    gchunk = 128
    for q in range(rw // gchunk):
        cp = pltpu.make_async_copy(
            emb_hbm.at[idx_v.at[pl.ds(q * gchunk, gchunk)]], rows_v, sem_g)
        cp.start()
        cp.wait()
        pltpu.sync_copy(
            rows_v, out_hbm.at[pl.ds(wid * rw + q * gchunk, gchunk)])


def kernel(class_logits, embedding):
    n, c = class_logits.shape
    _, d = embedding.shape
    rw = n // _NW
    ch = 2 * _L
    nchunk = rw // ch
    mesh = plsc.VectorSubcoreMesh(core_axis_name="c", subcore_axis_name="s")
    body = functools.partial(_sc_body, n, c, d, rw, ch, nchunk)
    f = pl.kernel(
        body,
        out_type=jax.ShapeDtypeStruct((n, d), jnp.float32),
        mesh=mesh,
        compiler_params=pltpu.CompilerParams(needs_layout_passes=False),
        scratch_types=[
            pltpu.VMEM((ch * c,), jnp.float32),
            pltpu.VMEM((ch * c,), jnp.float32),
            pltpu.VMEM((rw,), jnp.int32),
            pltpu.VMEM((128, d), jnp.float32),
            pltpu.SemaphoreType.DMA,
            pltpu.SemaphoreType.DMA,
            pltpu.SemaphoreType.DMA,
        ],
    )
    return f(class_logits.reshape(-1), embedding)


# speedup vs baseline: 1.6233x; 1.6233x over previous
import functools

import jax
import jax.numpy as jnp
from jax import lax
from jax.experimental import pallas as pl
from jax.experimental.pallas import tpu as pltpu
from jax.experimental.pallas import tpu_sc as plsc

_INFO = plsc.get_sparse_core_info()
_NC, _NS, _L = _INFO.num_cores, _INFO.num_subcores, _INFO.num_lanes
_NW = _NC * _NS


def _sc_body(n, c, d, rw, ch, nchunk, x_hbm, emb_hbm, out_hbm,
             x_a, x_b, idx_v, rows_v, sem_a, sem_b, sem_g):
    wid = lax.axis_index("s") * _NC + lax.axis_index("c")
    flat0 = wid * rw * c
    lanes = lax.iota(jnp.int32, _L)
    neg_inf = jnp.full((_L,), -jnp.inf, jnp.float32)
    zeros_i = jnp.zeros((_L,), jnp.int32)

    bufs = (x_a, x_b)
    sems = (sem_a, sem_b)

    def start_load(chunk, buf):
        return pltpu.make_async_copy(
            x_hbm.at[pl.ds(flat0 + chunk * ch * c, ch * c)],
            bufs[buf], sems[buf])

    start_load(0, 0).start()

    ngrp = ch // _L
    nstr = 4

    for chunk in range(nchunk):
        buf = chunk % 2
        if chunk + 1 < nchunk:
            start_load(chunk + 1, 1 - buf).start()
        start_load(chunk, buf).wait()

        base = [[(lanes * c + g * _L * c + s).astype(jnp.int32)
                 for s in range(nstr)] for g in range(ngrp)]
        cms = [neg_inf] * (ngrp * nstr)
        cas = [zeros_i] * (ngrp * nstr)
        ads = [base[g][s] for g in range(ngrp) for s in range(nstr)]
        step = jnp.full((_L,), nstr, jnp.int32)

        def jbody(t, carry, buf=buf):
            cms, cas, ads = (list(x) for x in carry)
            for s in range(nstr):
                for g in range(ngrp):
                    k = g * nstr + s
                    v = plsc.load_gather(bufs[buf], [ads[k]])
                    upd = v > cms[k]
                    cms[k] = jnp.where(upd, v, cms[k])
                    cas[k] = jnp.where(upd, ads[k], cas[k])
                    ads[k] = ads[k] + step
            return tuple(cms), tuple(cas), tuple(ads)

        cms, cas, _ = lax.fori_loop(
            0, c // nstr, jbody, (tuple(cms), tuple(cas), tuple(ads)))
        for g in range(ngrp):
            cm, ca = cms[g * nstr], cas[g * nstr]
            for s in range(1, nstr):
                cmb, cab = cms[g * nstr + s], cas[g * nstr + s]
                upd = (cmb > cm) | ((cmb == cm) & (cab < ca))
                cm = jnp.where(upd, cmb, cm)
                ca = jnp.where(upd, cab, ca)
            ci = ca - base[g][0]
            idx_v[pl.ds(chunk * ch + g * _L, _L)] = ci

    gchunk = 128
    for q in range(rw // gchunk):
        cp = pltpu.make_async_copy(
            emb_hbm.at[idx_v.at[pl.ds(q * gchunk, gchunk)]], rows_v, sem_g)
        cp.start()
        cp.wait()
        pltpu.sync_copy(
            rows_v, out_hbm.at[pl.ds(wid * rw + q * gchunk, gchunk)])


def kernel(class_logits, embedding):
    n, c = class_logits.shape
    _, d = embedding.shape
    rw = n // _NW
    ch = 2 * _L
    nchunk = rw // ch
    mesh = plsc.VectorSubcoreMesh(core_axis_name="c", subcore_axis_name="s")
    body = functools.partial(_sc_body, n, c, d, rw, ch, nchunk)
    f = pl.kernel(
        body,
        out_type=jax.ShapeDtypeStruct((n, d), jnp.float32),
        mesh=mesh,
        compiler_params=pltpu.CompilerParams(needs_layout_passes=False),
        scratch_types=[
            pltpu.VMEM((ch * c,), jnp.float32),
            pltpu.VMEM((ch * c,), jnp.float32),
            pltpu.VMEM((rw,), jnp.int32),
            pltpu.VMEM((128, d), jnp.float32),
            pltpu.SemaphoreType.DMA,
            pltpu.SemaphoreType.DMA,
            pltpu.SemaphoreType.DMA,
        ],
    )
    return f(class_logits.reshape(-1), embedding)


# speedup vs baseline: 1.6302x; 1.0042x over previous
import functools

import jax
import jax.numpy as jnp
from jax import lax
from jax.experimental import pallas as pl
from jax.experimental.pallas import tpu as pltpu
from jax.experimental.pallas import tpu_sc as plsc

_INFO = plsc.get_sparse_core_info()
_NC, _NS, _L = _INFO.num_cores, _INFO.num_subcores, _INFO.num_lanes
_NW = _NC * _NS


def _sc_body(n, c, d, rw, ch, nchunk, x_hbm, emb_hbm, out_hbm,
             x_a, x_b, idx_v, rows_v, sem_a, sem_b, sem_g):
    wid = lax.axis_index("s") * _NC + lax.axis_index("c")
    flat0 = wid * rw * c
    lanes = lax.iota(jnp.int32, _L)
    neg_inf = jnp.full((_L,), -jnp.inf, jnp.float32)
    zeros_i = jnp.zeros((_L,), jnp.int32)

    bufs = (x_a, x_b)
    sems = (sem_a, sem_b)

    def start_load(chunk, buf):
        return pltpu.make_async_copy(
            x_hbm.at[pl.ds(flat0 + chunk * ch * c, ch * c)],
            bufs[buf], sems[buf])

    start_load(0, 0).start()

    ngrp = ch // _L
    nstr = 4

    for chunk in range(nchunk):
        buf = chunk % 2
        if chunk + 1 < nchunk:
            start_load(chunk + 1, 1 - buf).start()
        start_load(chunk, buf).wait()

        base = [[(lanes * c + g * _L * c + s).astype(jnp.int32)
                 for s in range(nstr)] for g in range(ngrp)]
        cms = [neg_inf] * (ngrp * nstr)
        cas = [zeros_i] * (ngrp * nstr)
        ads = [base[g][s] for g in range(ngrp) for s in range(nstr)]
        step = jnp.full((_L,), nstr, jnp.int32)

        @plsc.parallel_loop(0, c // nstr, unroll=2,
                            carry=(tuple(cms), tuple(cas), tuple(ads)))
        def jloop(t, carry, buf=buf):
            cms, cas, ads = (list(x) for x in carry)
            for s in range(nstr):
                for g in range(ngrp):
                    k = g * nstr + s
                    v = plsc.load_gather(bufs[buf], [ads[k]])
                    upd = v > cms[k]
                    cms[k] = jnp.where(upd, v, cms[k])
                    cas[k] = jnp.where(upd, ads[k], cas[k])
                    ads[k] = ads[k] + step
            return tuple(cms), tuple(cas), tuple(ads)

        cms, cas, _ = jloop
        for g in range(ngrp):
            cm, ca = cms[g * nstr], cas[g * nstr]
            for s in range(1, nstr):
                cmb, cab = cms[g * nstr + s], cas[g * nstr + s]
                upd = (cmb > cm) | ((cmb == cm) & (cab < ca))
                cm = jnp.where(upd, cmb, cm)
                ca = jnp.where(upd, cab, ca)
            ci = ca - base[g][0]
            idx_v[pl.ds(chunk * ch + g * _L, _L)] = ci

    gchunk = 128
    for q in range(rw // gchunk):
        cp = pltpu.make_async_copy(
            emb_hbm.at[idx_v.at[pl.ds(q * gchunk, gchunk)]], rows_v, sem_g)
        cp.start()
        cp.wait()
        pltpu.sync_copy(
            rows_v, out_hbm.at[pl.ds(wid * rw + q * gchunk, gchunk)])


def kernel(class_logits, embedding):
    n, c = class_logits.shape
    _, d = embedding.shape
    rw = n // _NW
    ch = 2 * _L
    nchunk = rw // ch
    mesh = plsc.VectorSubcoreMesh(core_axis_name="c", subcore_axis_name="s")
    body = functools.partial(_sc_body, n, c, d, rw, ch, nchunk)
    f = pl.kernel(
        body,
        out_type=jax.ShapeDtypeStruct((n, d), jnp.float32),
        mesh=mesh,
        compiler_params=pltpu.CompilerParams(needs_layout_passes=False),
        scratch_types=[
            pltpu.VMEM((ch * c,), jnp.float32),
            pltpu.VMEM((ch * c,), jnp.float32),
            pltpu.VMEM((rw,), jnp.int32),
            pltpu.VMEM((128, d), jnp.float32),
            pltpu.SemaphoreType.DMA,
            pltpu.SemaphoreType.DMA,
            pltpu.SemaphoreType.DMA,
        ],
    )
    return f(class_logits.reshape(-1), embedding)
